# Initial kernel scaffold; baseline (speedup 1.0000x reference)
import jax
import jax.numpy as jnp
from jax.experimental import pallas as pl


def kernel(x, adj, to_pred, W1, b1, W2, b2, distmult):
    raise NotImplementedError("write your pallas kernel here")


# speedup vs baseline: 1.6484x; 1.6484x over previous
import functools

import jax
import jax.numpy as jnp
from jax import lax
from jax.experimental import pallas as pl
from jax.experimental.pallas import tpu as pltpu
from jax.experimental.pallas import tpu_sc as plsc

N, FEAT, HID, OUT = 10000, 256, 256, 128


BM = 400


def _s1_body(x_ref, w1_ref, o_ref):
    o_ref[...] = jnp.dot(x_ref[...], w1_ref[...],
                         preferred_element_type=jnp.float32)


def _layer1_body(adj_ref, s1_ref, b1_ref, w2_ref, o_ref):
    h = jnp.dot(adj_ref[...], s1_ref[...],
                preferred_element_type=jnp.float32)
    h = jnp.maximum(h + b1_ref[...], 0.0)
    o_ref[...] = jnp.dot(h, w2_ref[...],
                         preferred_element_type=jnp.float32)


def _layer2_body(adj_ref, s2_ref, b2_ref, dm_ref, e_ref, a_ref):
    e = jnp.dot(adj_ref[...], s2_ref[...],
                preferred_element_type=jnp.float32) + b2_ref[...]
    e_ref[...] = e
    a_ref[...] = e * dm_ref[...]


def _gcn_embeds(x, adj, W1, b1, W2, b2, distmult):
    s1 = pl.pallas_call(
        _s1_body,
        grid=(N // 2000,),
        in_specs=[
            pl.BlockSpec((2000, FEAT), lambda i: (i, 0)),
            pl.BlockSpec((FEAT, HID), lambda i: (0, 0)),
        ],
        out_specs=pl.BlockSpec((2000, HID), lambda i: (i, 0)),
        out_shape=jax.ShapeDtypeStruct((N, HID), jnp.float32),
    )(x, W1)

    s2 = pl.pallas_call(
        _layer1_body,
        grid=(N // BM,),
        in_specs=[
            pl.BlockSpec((BM, N), lambda i: (i, 0)),
            pl.BlockSpec((N, HID), lambda i: (0, 0)),
            pl.BlockSpec((1, HID), lambda i: (0, 0)),
            pl.BlockSpec((HID, OUT), lambda i: (0, 0)),
        ],
        out_specs=pl.BlockSpec((BM, OUT), lambda i: (i, 0)),
        out_shape=jax.ShapeDtypeStruct((N, OUT), jnp.float32),
    )(adj, s1, b1.reshape(1, HID), W2)

    e, a = pl.pallas_call(
        _layer2_body,
        grid=(N // BM,),
        in_specs=[
            pl.BlockSpec((BM, N), lambda i: (i, 0)),
            pl.BlockSpec((N, OUT), lambda i: (0, 0)),
            pl.BlockSpec((1, OUT), lambda i: (0, 0)),
            pl.BlockSpec((1, OUT), lambda i: (0, 0)),
        ],
        out_specs=[
            pl.BlockSpec((BM, OUT), lambda i: (i, 0)),
            pl.BlockSpec((BM, OUT), lambda i: (i, 0)),
        ],
        out_shape=[
            jax.ShapeDtypeStruct((N, OUT), jnp.float32),
            jax.ShapeDtypeStruct((N, OUT), jnp.float32),
        ],
    )(adj, s2, b2.reshape(1, OUT), distmult.reshape(1, OUT))
    return e, a


NW = 32


def _lane_perm(x, idx):
    dn = lax.GatherDimensionNumbers(
        offset_dims=(), collapsed_slice_dims=(0,), start_index_map=(0,))
    return lax.gather(x, idx[:, None], dn, (1,),
                      mode=lax.GatherScatterMode.PROMISE_IN_BOUNDS)
CHUNK = 112
GRP = CHUNK // 16


def _score_body(a_hbm, e_hbm, isrc_hbm, idst_hbm, out_hbm,
                isrc_v, idst_v, rs_v, rd_v, out_v, sem_s, sem_d):
    wid = lax.axis_index("s") * 2 + lax.axis_index("c")
    n_chunks = isrc_hbm.shape[0] // (NW * CHUNK)
    base = wid * (n_chunks * CHUNK)
    lane = lax.broadcasted_iota(jnp.int32, (16,), 0)

    def chunk_body(j, carry):
        cb = base + j * CHUNK
        pltpu.sync_copy(isrc_hbm.at[pl.ds(cb, CHUNK)], isrc_v)
        pltpu.sync_copy(idst_hbm.at[pl.ds(cb, CHUNK)], idst_v)
        cp_s = pltpu.async_copy(a_hbm.at[isrc_v], rs_v, sem_s)
        cp_d = pltpu.async_copy(e_hbm.at[idst_v], rd_v, sem_d)
        cp_s.wait()
        cp_d.wait()

        def group_body(g, carry2):
            out_vec = jnp.zeros((16,), jnp.float32)
            for t in range(16):
                c = g * 16 + t
                acc = rs_v[c, pl.ds(0, 16)] * rd_v[c, pl.ds(0, 16)]
                for v in range(1, OUT // 16):
                    acc = acc + (rs_v[c, pl.ds(v * 16, 16)] *
                                 rd_v[c, pl.ds(v * 16, 16)])
                for sh in (8, 4, 2, 1):
                    acc = acc + _lane_perm(acc, lane ^ sh)
                out_vec = jnp.where(lane == t, acc, out_vec)
            out_v[pl.ds(g * 16, 16)] = out_vec
            return carry2

        lax.fori_loop(0, GRP, group_body, 0)
        pltpu.sync_copy(out_v, out_hbm.at[pl.ds(cb, CHUNK)])
        return carry

    lax.fori_loop(0, n_chunks, chunk_body, 0)


def _score(a, e, isrc, idst, p_pad):
    mesh = plsc.VectorSubcoreMesh(core_axis_name="c", subcore_axis_name="s")
    f = functools.partial(
        pl.kernel,
        mesh=mesh,
        out_type=jax.ShapeDtypeStruct((p_pad,), jnp.float32),
        scratch_types=[
            pltpu.VMEM((CHUNK,), jnp.int32),
            pltpu.VMEM((CHUNK,), jnp.int32),
            pltpu.VMEM((CHUNK, OUT), jnp.float32),
            pltpu.VMEM((CHUNK, OUT), jnp.float32),
            pltpu.VMEM((CHUNK,), jnp.float32),
            pltpu.SemaphoreType.DMA,
            pltpu.SemaphoreType.DMA,
        ],
    )(_score_body)
    return f(a, e, isrc, idst)


def kernel(x, adj, to_pred, W1, b1, W2, b2, distmult):
    p = to_pred.shape[0]
    per_w = ((p + NW * CHUNK - 1) // (NW * CHUNK)) * CHUNK
    p_pad = NW * per_w
    tp = jnp.pad(to_pred, ((0, p_pad - p), (0, 0)))
    isrc = tp[:, 0]
    idst = tp[:, 1]

    e, a = _gcn_embeds(x, adj, W1, b1, W2, b2, distmult)
    dot = _score(a, e, isrc, idst, p_pad)
    return dot[:p]


# speedup vs baseline: 1.9566x; 1.1869x over previous
import functools

import jax
import jax.numpy as jnp
from jax import lax
from jax.experimental import pallas as pl
from jax.experimental.pallas import tpu as pltpu
from jax.experimental.pallas import tpu_sc as plsc

N, FEAT, HID, OUT = 10000, 256, 256, 128


BM = 400


def _s1_body(x_ref, w1_ref, o_ref):
    o_ref[...] = jnp.dot(x_ref[...], w1_ref[...],
                         preferred_element_type=jnp.float32)


def _layer1_body(adj_ref, s1_ref, b1_ref, w2_ref, o_ref):
    h = jnp.dot(adj_ref[...], s1_ref[...],
                preferred_element_type=jnp.float32)
    h = jnp.maximum(h + b1_ref[...], 0.0)
    o_ref[...] = jnp.dot(h, w2_ref[...],
                         preferred_element_type=jnp.float32)


def _layer2_body(adj_ref, s2_ref, b2_ref, dm_ref, e_ref, a_ref):
    e = jnp.dot(adj_ref[...], s2_ref[...],
                preferred_element_type=jnp.float32) + b2_ref[...]
    e_ref[...] = e
    a_ref[...] = e * dm_ref[...]


def _gcn_embeds(x, adj, W1, b1, W2, b2, distmult):
    s1 = pl.pallas_call(
        _s1_body,
        grid=(N // 2000,),
        in_specs=[
            pl.BlockSpec((2000, FEAT), lambda i: (i, 0)),
            pl.BlockSpec((FEAT, HID), lambda i: (0, 0)),
        ],
        out_specs=pl.BlockSpec((2000, HID), lambda i: (i, 0)),
        out_shape=jax.ShapeDtypeStruct((N, HID), jnp.float32),
    )(x, W1)

    s2 = pl.pallas_call(
        _layer1_body,
        grid=(N // BM,),
        in_specs=[
            pl.BlockSpec((BM, N), lambda i: (i, 0)),
            pl.BlockSpec((N, HID), lambda i: (0, 0)),
            pl.BlockSpec((1, HID), lambda i: (0, 0)),
            pl.BlockSpec((HID, OUT), lambda i: (0, 0)),
        ],
        out_specs=pl.BlockSpec((BM, OUT), lambda i: (i, 0)),
        out_shape=jax.ShapeDtypeStruct((N, OUT), jnp.float32),
    )(adj, s1, b1.reshape(1, HID), W2)

    e, a = pl.pallas_call(
        _layer2_body,
        grid=(N // BM,),
        in_specs=[
            pl.BlockSpec((BM, N), lambda i: (i, 0)),
            pl.BlockSpec((N, OUT), lambda i: (0, 0)),
            pl.BlockSpec((1, OUT), lambda i: (0, 0)),
            pl.BlockSpec((1, OUT), lambda i: (0, 0)),
        ],
        out_specs=[
            pl.BlockSpec((BM, OUT), lambda i: (i, 0)),
            pl.BlockSpec((BM, OUT), lambda i: (i, 0)),
        ],
        out_shape=[
            jax.ShapeDtypeStruct((N, OUT), jnp.float32),
            jax.ShapeDtypeStruct((N, OUT), jnp.float32),
        ],
    )(adj, s2, b2.reshape(1, OUT), distmult.reshape(1, OUT))
    return e, a


NW = 32


def _lane_perm(x, idx):
    dn = lax.GatherDimensionNumbers(
        offset_dims=(), collapsed_slice_dims=(0,), start_index_map=(0,))
    return lax.gather(x, idx[:, None], dn, (1,),
                      mode=lax.GatherScatterMode.PROMISE_IN_BOUNDS)
CHUNK = 112
GRP = CHUNK // 16


def _score_body(a_hbm, e_hbm, isrc_hbm, idst_hbm, out_hbm,
                isrc_v, idst_v, rs_v, rd_v, out_v, sem0, sem1):
    wid = lax.axis_index("s") * 2 + lax.axis_index("c")
    n_chunks = isrc_hbm.shape[1]
    lane = lax.broadcasted_iota(jnp.int32, (16,), 0)
    sems = (sem0, sem1)

    pltpu.sync_copy(isrc_hbm.at[wid], isrc_v)
    pltpu.sync_copy(idst_hbm.at[wid], idst_v)

    def fire(j, b):
        cps = pltpu.async_copy(a_hbm.at[isrc_v.at[j]], rs_v.at[b], sems[b])
        cpd = pltpu.async_copy(e_hbm.at[idst_v.at[j]], rd_v.at[b], sems[b])
        return cps, cpd

    def drain(b):
        pltpu.make_async_copy(a_hbm.at[isrc_v.at[0]], rs_v.at[b],
                              sems[b]).wait()
        pltpu.make_async_copy(e_hbm.at[idst_v.at[0]], rd_v.at[b],
                              sems[b]).wait()

    def compute(j, b):
        def group_body(g, carry2):
            out_vec = jnp.zeros((16,), jnp.float32)
            for t in range(16):
                c = g * 16 + t
                acc = rs_v[b, c, pl.ds(0, 16)] * rd_v[b, c, pl.ds(0, 16)]
                for v in range(1, OUT // 16):
                    acc = acc + (rs_v[b, c, pl.ds(v * 16, 16)] *
                                 rd_v[b, c, pl.ds(v * 16, 16)])
                for sh in (8, 4, 2, 1):
                    acc = acc + _lane_perm(acc, lane ^ sh)
                out_vec = jnp.where(lane == t, acc, out_vec)
            out_v[j, pl.ds(g * 16, 16)] = out_vec
            return carry2

        lax.fori_loop(0, GRP, group_body, 0)

    fire(0, 0)
    fire(1, 1)

    def pipe_body(jj, carry):
        j = 2 * jj
        drain(0)
        compute(j, 0)

        @pl.when(j + 2 < n_chunks)
        def _():
            fire(j + 2, 0)

        drain(1)
        compute(j + 1, 1)

        @pl.when(j + 3 < n_chunks)
        def _():
            fire(j + 3, 1)

        return carry

    lax.fori_loop(0, n_chunks // 2, pipe_body, 0)
    pltpu.sync_copy(out_v, out_hbm.at[wid])


def _score(a, e, isrc, idst, n_chunks):
    mesh = plsc.VectorSubcoreMesh(core_axis_name="c", subcore_axis_name="s")
    f = functools.partial(
        pl.kernel,
        mesh=mesh,
        out_type=jax.ShapeDtypeStruct((NW, n_chunks, CHUNK), jnp.float32),
        scratch_types=[
            pltpu.VMEM((n_chunks, CHUNK), jnp.int32),
            pltpu.VMEM((n_chunks, CHUNK), jnp.int32),
            pltpu.VMEM((2, CHUNK, OUT), jnp.float32),
            pltpu.VMEM((2, CHUNK, OUT), jnp.float32),
            pltpu.VMEM((n_chunks, CHUNK), jnp.float32),
            pltpu.SemaphoreType.DMA,
            pltpu.SemaphoreType.DMA,
        ],
    )(_score_body)
    return f(a, e, isrc, idst)


def kernel(x, adj, to_pred, W1, b1, W2, b2, distmult):
    p = to_pred.shape[0]
    per_w = ((p + NW * CHUNK - 1) // (NW * CHUNK)) * CHUNK
    n_chunks = per_w // CHUNK
    if n_chunks % 2:
        n_chunks += 1
        per_w = n_chunks * CHUNK
    p_pad = NW * per_w
    tp = jnp.pad(to_pred, ((0, p_pad - p), (0, 0)))
    isrc = tp[:, 0].reshape(NW, n_chunks, CHUNK)
    idst = tp[:, 1].reshape(NW, n_chunks, CHUNK)

    e, a = _gcn_embeds(x, adj, W1, b1, W2, b2, distmult)
    dot = _score(a, e, isrc, idst, n_chunks)
    return dot.reshape(p_pad)[:p]
